# Initial kernel scaffold; baseline (speedup 1.0000x reference)
import jax
import jax.numpy as jnp
from jax.experimental import pallas as pl


def kernel(x, fc_w, fc_b):
    raise NotImplementedError("write your pallas kernel here")


# speedup vs baseline: 2.2988x; 2.2988x over previous
import functools

import jax
import jax.numpy as jnp
from jax import lax
from jax.experimental import pallas as pl
from jax.experimental.pallas import tpu as pltpu
from jax.experimental.pallas import tpu_sc as plsc

_B, _H, _W, _C = 32, 32, 32, 768
_N = _H * _W
_K = 1000
_KP = 1024
_SEL = 128
_OUT_EMB = 1024
_L = 16


def _scorer_body(x_ref, w_ref, b_ref, mean_ref, score_ref):
    xb = x_ref[0]
    pred = jnp.dot(xb, w_ref[...], preferred_element_type=jnp.float32)
    pred = pred + b_ref[...]
    m = jnp.max(pred, axis=1, keepdims=True)
    s = jnp.sum(jnp.exp(pred - m), axis=1)
    score_ref[...] = (1.0 / s)[None, None, :]
    mean_ref[...] = (jnp.sum(pred, axis=0) * (1.0 / _N))[None, None, :]


def _run_scorer(xf, w_pad, b_pad, interpret=False):
    return pl.pallas_call(
        _scorer_body,
        grid=(_B,),
        in_specs=[
            pl.BlockSpec((1, _N, _C), lambda b: (b, 0, 0)),
            pl.BlockSpec((_C, _KP), lambda b: (0, 0)),
            pl.BlockSpec((1, _KP), lambda b: (0, 0)),
        ],
        out_specs=[
            pl.BlockSpec((1, 1, _KP), lambda b: (b, 0, 0)),
            pl.BlockSpec((1, 1, _N), lambda b: (b, 0, 0)),
        ],
        out_shape=[
            jax.ShapeDtypeStruct((_B, 1, _KP), jnp.float32),
            jax.ShapeDtypeStruct((_B, 1, _N), jnp.float32),
        ],
        interpret=interpret,
    )(xf, w_pad, b_pad)


def _splat(v):
    return jnp.broadcast_to(v, (_L,)) if v.ndim == 0 else v


def _sc_body(score_hbm, x_hbm, out_hbm, score_v, idx_v, rows_v, sem):
    nc = 2
    b = lax.axis_index("s") * nc + lax.axis_index("c")
    pltpu.sync_copy(score_hbm.at[b], score_v)

    iota = lax.iota(jnp.int32, _L)
    lane0 = iota == 0
    neg = jnp.full((_L,), -jnp.inf, jnp.float32)

    gmax = neg
    for g in range(16):
        s0 = score_v[pl.ds(64 * g + 0, _L)]
        s1 = score_v[pl.ds(64 * g + 16, _L)]
        s2 = score_v[pl.ds(64 * g + 32, _L)]
        s3 = score_v[pl.ds(64 * g + 48, _L)]
        mg = jnp.max(jnp.maximum(jnp.maximum(s0, s1), jnp.maximum(s2, s3)))
        gmax = jnp.where(iota == g, mg, gmax)

    base = b * _N

    def body(it, gmax):
        big = jnp.max(gmax)
        mv = jnp.full((_L,), big)
        grp = _splat(plsc.all_reduce_ffs(gmax == mv))
        q0 = plsc.load_gather(score_v, [grp * 64 + iota])
        q1 = plsc.load_gather(score_v, [grp * 64 + 16 + iota])
        q2 = plsc.load_gather(score_v, [grp * 64 + 32 + iota])
        q3 = plsc.load_gather(score_v, [grp * 64 + 48 + iota])
        e0, e1, e2, e3 = q0 == mv, q1 == mv, q2 == mv, q3 == mv
        c0 = _splat(plsc.all_reduce_population_count(e0))
        c1 = _splat(plsc.all_reduce_population_count(e1))
        c2 = _splat(plsc.all_reduce_population_count(e2))
        f0 = _splat(plsc.all_reduce_ffs(e0))
        f1 = _splat(plsc.all_reduce_ffs(e1))
        f2 = _splat(plsc.all_reduce_ffs(e2))
        f3 = _splat(plsc.all_reduce_ffs(e3))
        h0 = c0 > 0
        h1 = jnp.logical_and(jnp.logical_not(h0), c1 > 0)
        h2 = jnp.logical_and(jnp.logical_not(jnp.logical_or(h0, h1)), c2 > 0)
        h3 = jnp.logical_not(jnp.logical_or(jnp.logical_or(h0, h1), h2))
        off = jnp.where(h0, 0, jnp.where(h1, 16, jnp.where(h2, 32, 48)))
        lane = jnp.where(h0, f0, jnp.where(h1, f1, jnp.where(h2, f2, f3)))
        idx = grp * 64 + off + lane
        it_v = jnp.full((_L,), it, jnp.int32)
        plsc.store_scatter(idx_v, [it_v], idx + base, mask=lane0)
        plsc.store_scatter(score_v, [idx], neg, mask=lane0)
        q0 = jnp.where(jnp.logical_and(h0, iota == lane), neg, q0)
        q1 = jnp.where(jnp.logical_and(h1, iota == lane), neg, q1)
        q2 = jnp.where(jnp.logical_and(h2, iota == lane), neg, q2)
        q3 = jnp.where(jnp.logical_and(h3, iota == lane), neg, q3)
        ng = jnp.max(jnp.maximum(jnp.maximum(q0, q1), jnp.maximum(q2, q3)))
        return jnp.where(iota == grp, ng, gmax)

    lax.fori_loop(0, _SEL, body, gmax)

    pltpu.async_copy(x_hbm.at[idx_v], rows_v, sem).wait()
    pltpu.sync_copy(rows_v, out_hbm.at[b])


def _run_select_gather(scores, xflat):
    mesh = plsc.VectorSubcoreMesh(core_axis_name="c", subcore_axis_name="s")
    kern = pl.kernel(
        _sc_body,
        out_type=jax.ShapeDtypeStruct((_B, _SEL, _C), jnp.float32),
        mesh=mesh,
        scratch_types=[
            pltpu.VMEM((_N,), jnp.float32),
            pltpu.VMEM((_SEL,), jnp.int32),
            pltpu.VMEM((_SEL, _C), jnp.float32),
            pltpu.SemaphoreType.DMA,
        ],
        compiler_params=pltpu.CompilerParams(needs_layout_passes=False),
    )
    return kern(scores, xflat)


@jax.jit
def kernel(x, fc_w, fc_b):
    xf = x.reshape(_B, _N, _C)
    w_pad = jnp.zeros((_C, _KP), jnp.float32).at[:, :_K].set(fc_w.T)
    b_pad = jnp.full((1, _KP), -1e30, jnp.float32).at[0, :_K].set(fc_b)
    mean_pad, scores = _run_scorer(xf, w_pad, b_pad)
    mean_pad = mean_pad.reshape(_B, _KP)
    scores = scores.reshape(_B, _N)
    mean_pred = mean_pad[:, :_K]
    selected = _run_select_gather(scores, xf.reshape(_B * _N, _C))
    padded = jnp.pad(selected, ((0, 0), (0, 0), (0, _OUT_EMB - _C)))
    return (mean_pred, padded)


# speedup vs baseline: 2.4931x; 1.0846x over previous
import functools

import jax
import jax.numpy as jnp
from jax import lax
from jax.experimental import pallas as pl
from jax.experimental.pallas import tpu as pltpu
from jax.experimental.pallas import tpu_sc as plsc

_B, _H, _W, _C = 32, 32, 32, 768
_N = _H * _W
_K = 1000
_KP = 1024
_SEL = 128
_OUT_EMB = 1024
_L = 16


def _scorer_body(x_ref, w_ref, b_ref, mean_ref, score_ref):
    xb = x_ref[0]
    pred = lax.dot_general(xb, w_ref[...], (((1,), (1,)), ((), ())),
                           preferred_element_type=jnp.float32)
    pred = pred + b_ref[...]
    m = jnp.max(pred, axis=1, keepdims=True)
    s = jnp.sum(jnp.exp(pred - m), axis=1)
    score_ref[...] = (1.0 / s)[None, None, :]
    mean_ref[...] = (jnp.sum(pred, axis=0) * (1.0 / _N))[None, None, :]


def _run_scorer(xf, w, b_row, interpret=False):
    return pl.pallas_call(
        _scorer_body,
        grid=(_B,),
        in_specs=[
            pl.BlockSpec((1, _N, _C), lambda b: (b, 0, 0)),
            pl.BlockSpec((_K, _C), lambda b: (0, 0)),
            pl.BlockSpec((1, _K), lambda b: (0, 0)),
        ],
        out_specs=[
            pl.BlockSpec((1, 1, _K), lambda b: (b, 0, 0)),
            pl.BlockSpec((1, 1, _N), lambda b: (b, 0, 0)),
        ],
        out_shape=[
            jax.ShapeDtypeStruct((_B, 1, _K), jnp.float32),
            jax.ShapeDtypeStruct((_B, 1, _N), jnp.float32),
        ],
        interpret=interpret,
    )(xf, w, b_row)


def _splat(v):
    return jnp.broadcast_to(v, (_L,)) if v.ndim == 0 else v


def _sc_body(score_hbm, x_hbm, zero_hbm, out_hbm, score_v, idx_v, rows_v, zero_v, sem):
    nc = 2
    b = lax.axis_index("s") * nc + lax.axis_index("c")
    pltpu.sync_copy(score_hbm.at[b], score_v)
    pltpu.sync_copy(zero_hbm, zero_v)

    iota = lax.iota(jnp.int32, _L)
    lane0 = iota == 0
    neg = jnp.full((_L,), -jnp.inf, jnp.float32)

    gmax = neg
    for g in range(16):
        s0 = score_v[pl.ds(64 * g + 0, _L)]
        s1 = score_v[pl.ds(64 * g + 16, _L)]
        s2 = score_v[pl.ds(64 * g + 32, _L)]
        s3 = score_v[pl.ds(64 * g + 48, _L)]
        mg = jnp.max(jnp.maximum(jnp.maximum(s0, s1), jnp.maximum(s2, s3)))
        gmax = jnp.where(iota == g, mg, gmax)

    base = b * _N

    def body(it, gmax):
        big = jnp.max(gmax)
        mv = jnp.full((_L,), big)
        grp = _splat(plsc.all_reduce_ffs(gmax == mv))
        q0 = plsc.load_gather(score_v, [grp * 64 + iota])
        q1 = plsc.load_gather(score_v, [grp * 64 + 16 + iota])
        q2 = plsc.load_gather(score_v, [grp * 64 + 32 + iota])
        q3 = plsc.load_gather(score_v, [grp * 64 + 48 + iota])
        e0, e1, e2, e3 = q0 == mv, q1 == mv, q2 == mv, q3 == mv
        c0 = _splat(plsc.all_reduce_population_count(e0))
        c1 = _splat(plsc.all_reduce_population_count(e1))
        c2 = _splat(plsc.all_reduce_population_count(e2))
        f0 = _splat(plsc.all_reduce_ffs(e0))
        f1 = _splat(plsc.all_reduce_ffs(e1))
        f2 = _splat(plsc.all_reduce_ffs(e2))
        f3 = _splat(plsc.all_reduce_ffs(e3))
        h0 = c0 > 0
        h1 = jnp.logical_and(jnp.logical_not(h0), c1 > 0)
        h2 = jnp.logical_and(jnp.logical_not(jnp.logical_or(h0, h1)), c2 > 0)
        h3 = jnp.logical_not(jnp.logical_or(jnp.logical_or(h0, h1), h2))
        off = jnp.where(h0, 0, jnp.where(h1, 16, jnp.where(h2, 32, 48)))
        lane = jnp.where(h0, f0, jnp.where(h1, f1, jnp.where(h2, f2, f3)))
        idx = grp * 64 + off + lane
        it_v = jnp.full((_L,), it, jnp.int32)
        plsc.store_scatter(idx_v, [it_v], idx + base, mask=lane0)
        plsc.store_scatter(score_v, [idx], neg, mask=lane0)
        q0 = jnp.where(jnp.logical_and(h0, iota == lane), neg, q0)
        q1 = jnp.where(jnp.logical_and(h1, iota == lane), neg, q1)
        q2 = jnp.where(jnp.logical_and(h2, iota == lane), neg, q2)
        q3 = jnp.where(jnp.logical_and(h3, iota == lane), neg, q3)
        ng = jnp.max(jnp.maximum(jnp.maximum(q0, q1), jnp.maximum(q2, q3)))
        return jnp.where(iota == grp, ng, gmax)

    lax.fori_loop(0, _SEL, body, gmax)

    pltpu.async_copy(x_hbm.at[idx_v], rows_v, sem).wait()
    pltpu.sync_copy(rows_v, out_hbm.at[b, :, pl.ds(0, _C)])
    pltpu.sync_copy(zero_v, out_hbm.at[b, pl.ds(0, 64), pl.ds(_C, _OUT_EMB - _C)])
    pltpu.sync_copy(zero_v, out_hbm.at[b, pl.ds(64, 64), pl.ds(_C, _OUT_EMB - _C)])


def _run_select_gather(scores, xflat, zeros):
    mesh = plsc.VectorSubcoreMesh(core_axis_name="c", subcore_axis_name="s")
    kern = pl.kernel(
        _sc_body,
        out_type=jax.ShapeDtypeStruct((_B, _SEL, _OUT_EMB), jnp.float32),
        mesh=mesh,
        scratch_types=[
            pltpu.VMEM((_N,), jnp.float32),
            pltpu.VMEM((_SEL,), jnp.int32),
            pltpu.VMEM((_SEL, _C), jnp.float32),
            pltpu.VMEM((64, _OUT_EMB - _C), jnp.float32),
            pltpu.SemaphoreType.DMA,
        ],
        compiler_params=pltpu.CompilerParams(needs_layout_passes=False),
    )
    return kern(scores, xflat, zeros)


@jax.jit
def kernel(x, fc_w, fc_b):
    xf = x.reshape(_B, _N, _C)
    mean_out, score_out = _run_scorer(xf, fc_w, fc_b.reshape(1, _K))
    mean_pred = mean_out.reshape(_B, _K)
    scores = score_out.reshape(_B, _N)
    zeros = jnp.zeros((64, _OUT_EMB - _C), jnp.float32)
    padded = _run_select_gather(scores, xf.reshape(_B * _N, _C), zeros)
    return (mean_pred, padded)
